# baseline (speedup 1.0000x reference)
import functools

import jax
import jax.numpy as jnp
from jax import lax
from jax.experimental import pallas as pl
from jax.experimental.pallas import tpu as pltpu, tpu_sc as plsc

N = 10000
D = 128
E = 320000
G = 128

NC = 2
NS = 16
NW = NC * NS
CH = 128
CPT = 80
E_PAD = NW * CPT * CH
NPAD = 10112
RPT = NPAD // NS

def _sc_mesh():
    return plsc.VectorSubcoreMesh(core_axis_name="c", subcore_axis_name="s",
                                  num_cores=NC, num_subcores=NS)


def _wid():
    return lax.axis_index("s") * NC + lax.axis_index("c")


def _sc_deg_body(dst_hbm, ones_hbm, zeros_hbm, out_hbm, ones_v, idx_v, acc, sem):
    cid = lax.axis_index("c")
    sid = lax.axis_index("s")
    wid = _wid()
    pltpu.sync_copy(zeros_hbm, acc.at[pl.ds(sid * RPT, RPT)])
    pltpu.sync_copy(ones_hbm, ones_v)
    plsc.subcore_barrier()
    idx0, idx1 = idx_v
    sem0, sem1 = sem

    @pl.loop(0, CPT // 2)
    def _pair(t):
        base = wid * (CPT * CH) + t * (2 * CH)
        pltpu.sync_copy(dst_hbm.at[pl.ds(base, CH)], idx0)
        d0 = pltpu.async_copy(ones_v, acc.at[idx0], sem0, add=True)
        pltpu.sync_copy(dst_hbm.at[pl.ds(base + CH, CH)], idx1)
        d1 = pltpu.async_copy(ones_v, acc.at[idx1], sem1, add=True)
        d0.wait()
        d1.wait()

    plsc.subcore_barrier()
    pltpu.sync_copy(acc.at[pl.ds(sid * RPT, RPT)],
                    out_hbm.at[cid, pl.ds(sid * RPT, RPT)])


def _sc_deg(dst_pad, onesD, zerosD):
    return pl.kernel(
        _sc_deg_body,
        out_type=jax.ShapeDtypeStruct((NC, NPAD, D), jnp.float32),
        mesh=_sc_mesh(),
        scratch_types=[
            pltpu.VMEM((CH, D), jnp.float32),
            [pltpu.VMEM((CH,), jnp.int32)] * 2,
            pltpu.VMEM_SHARED((NPAD, D), jnp.float32),
            [pltpu.SemaphoreType.DMA] * 2,
        ],
    )(dst_pad, onesD, zerosD)


_NBUF = 2


_CPT0 = 160
_CPT1 = 0


def _sc_spmm_body(g_hbm, src_hbm, dst_hbm, zeros_hbm, out_hbm,
                  idx_s, idx_d, rows, acc, sems):
    cid = lax.axis_index("c")
    sid = lax.axis_index("s")
    npairs = lax.select(cid == 0, _CPT0 // 2, _CPT1 // 2)
    ebase = lax.select(cid == 0, sid * (_CPT0 * CH),
                       NS * (_CPT0 * CH) + sid * (_CPT1 * CH))
    pltpu.sync_copy(zeros_hbm, acc.at[pl.ds(sid * RPT, RPT)])

    def start(b, c):
        pltpu.sync_copy(src_hbm.at[pl.ds(ebase + c * CH, CH)], idx_s[b])
        pltpu.async_copy(g_hbm.at[idx_s[b]], rows[b], sems[b])

    def finish(b, c):
        pltpu.sync_copy(dst_hbm.at[pl.ds(ebase + c * CH, CH)], idx_d[b])
        pltpu.make_async_copy(g_hbm.at[idx_s[b]], rows[b], sems[b]).wait()
        pltpu.sync_copy(rows[b], acc.at[idx_d[b]], add=True)

    plsc.subcore_barrier()

    @pl.when(npairs > 0)
    def _prime():
        start(0, 0)

    @pl.loop(0, npairs)
    def _pair(t):
        c = t * 2
        start(1, c + 1)
        finish(0, c)

        @pl.when(t < npairs - 1)
        def _prefetch():
            start(0, c + 2)

        finish(1, c + 1)

    plsc.subcore_barrier()
    pltpu.sync_copy(acc.at[pl.ds(sid * RPT, RPT)],
                    out_hbm.at[cid, pl.ds(sid * RPT, RPT)])


def _sc_spmm(g, src_pad, dst_pad, zerosD):
    return pl.kernel(
        _sc_spmm_body,
        out_type=jax.ShapeDtypeStruct((NC, NPAD, D), jnp.float32),
        mesh=_sc_mesh(),
        scratch_types=[
            [pltpu.VMEM((CH,), jnp.int32)] * _NBUF,
            [pltpu.VMEM((CH,), jnp.int32)] * _NBUF,
            [pltpu.VMEM((CH, D), jnp.float32)] * _NBUF,
            pltpu.VMEM_SHARED((NPAD, D), jnp.float32),
            [pltpu.SemaphoreType.DMA] * _NBUF,
        ],
    )(g, src_pad, dst_pad, zerosD)


_BLK = 1000
_GRID = N // _BLK

_row_spec = pl.BlockSpec((_BLK, D), lambda i: (i, 0))
_col_spec = pl.BlockSpec((_BLK, 1), lambda i: (i, 0))
_w_spec = pl.BlockSpec((D, D), lambda i: (0, 0))
_b_spec = pl.BlockSpec((1, D), lambda i: (0, 0))


def _tc1_body(x_ref, w_ref, dega_ref, degb_ref, g_ref, dinv_ref):
    deg = dega_ref[...] + degb_ref[...] + 1.0
    dinv = lax.rsqrt(deg)
    dinv_ref[...] = dinv
    h = jnp.dot(x_ref[...], w_ref[...], preferred_element_type=jnp.float32)
    g_ref[...] = dinv * h


def _tc1(x, W1, dega, degb):
    return pl.pallas_call(
        _tc1_body,
        grid=(_GRID,),
        in_specs=[_row_spec, _w_spec, _col_spec, _col_spec],
        out_specs=[_row_spec, _col_spec],
        out_shape=[jax.ShapeDtypeStruct((N, D), jnp.float32),
                   jax.ShapeDtypeStruct((N, 1), jnp.float32)],
    )(x, W1, dega, degb)


def _tc_mid_body(sa_ref, sb_ref, g_ref, dinv_ref, b_ref, w_ref, gout_ref):
    dinv = dinv_ref[...]
    t = dinv * (sa_ref[...] + sb_ref[...] + g_ref[...]) + b_ref[...]
    t = jnp.maximum(t, 0.0)
    gout_ref[...] = dinv * jnp.dot(t, w_ref[...],
                                   preferred_element_type=jnp.float32)


def _tc_mid(sa, sb, g, dinv, b, Wn):
    return pl.pallas_call(
        _tc_mid_body,
        grid=(_GRID,),
        in_specs=[_row_spec, _row_spec, _row_spec, _col_spec, _b_spec, _w_spec],
        out_specs=_row_spec,
        out_shape=jax.ShapeDtypeStruct((N, D), jnp.float32),
    )(sa, sb, g, dinv, b, Wn)


def _tc_pool_body(sa_ref, sb_ref, g_ref, dinv_ref, b_ref, batch_ref, out_ref,
                  accp, accc):
    i = pl.program_id(0)

    @pl.when(i == 0)
    def _init():
        accp[...] = jnp.zeros_like(accp)
        accc[...] = jnp.zeros_like(accc)

    h = dinv_ref[...] * (sa_ref[...] + sb_ref[...] + g_ref[...]) + b_ref[...]
    h = jnp.maximum(h, 0.0)
    gids = lax.broadcasted_iota(jnp.int32, (_BLK, G), 1)
    oh = (batch_ref[...] == gids).astype(jnp.float32)
    dims = (((0,), (0,)), ((), ()))
    accp[...] += lax.dot_general(oh, h, dims,
                                 preferred_element_type=jnp.float32)
    accc[...] += lax.dot_general(oh, jnp.ones_like(h), dims,
                                 preferred_element_type=jnp.float32)

    @pl.when(i == _GRID - 1)
    def _fin():
        out_ref[...] = accp[...] / jnp.maximum(accc[...], 1.0)


def _tc_pool(sa, sb, g, dinv, b, batch2):
    return pl.pallas_call(
        _tc_pool_body,
        grid=(_GRID,),
        in_specs=[_row_spec, _row_spec, _row_spec, _col_spec, _b_spec,
                  pl.BlockSpec((_BLK, 1), lambda i: (i, 0))],
        out_specs=pl.BlockSpec((G, D), lambda i: (0, 0)),
        out_shape=jax.ShapeDtypeStruct((G, D), jnp.float32),
        scratch_shapes=[pltpu.VMEM((G, D), jnp.float32),
                        pltpu.VMEM((G, D), jnp.float32)],
    )(sa, sb, g, dinv, b, batch2)


def kernel(x, edge_index, batch, W1, b1, W2, b2, W3, b3, Wl, bl):
    del Wl, bl
    src = edge_index[0].astype(jnp.int32)
    dst = edge_index[1].astype(jnp.int32)
    pad = E_PAD - E
    src_pad = jnp.concatenate([src, jnp.zeros((pad,), jnp.int32)])
    dst_pad = jnp.concatenate([dst, jnp.full((pad,), N, jnp.int32)])
    onesD = jnp.ones((CH, D), jnp.float32)
    zerosD = jnp.zeros((RPT, D), jnp.float32)
    batch2 = batch.astype(jnp.int32)[:, None]

    deg = _sc_deg(dst_pad, onesD, zerosD)
    dega = deg[0, :N, 0:1]
    degb = deg[1, :N, 0:1]

    g1, dinv = _tc1(x, W1, dega, degb)
    s1 = _sc_spmm(g1, src_pad, dst_pad, zerosD)
    g2 = _tc_mid(s1[0, :N], s1[1, :N], g1, dinv, b1[None, :], W2)
    s2 = _sc_spmm(g2, src_pad, dst_pad, zerosD)
    g3 = _tc_mid(s2[0, :N], s2[1, :N], g2, dinv, b2[None, :], W3)
    s3 = _sc_spmm(g3, src_pad, dst_pad, zerosD)
    pooled = _tc_pool(s3[0, :N], s3[1, :N], g3, dinv, b3[None, :], batch2)
    return pooled


# speedup vs baseline: 2.4364x; 2.4364x over previous
import jax
import jax.numpy as jnp
from jax import lax
from jax.experimental import pallas as pl
from jax.experimental.pallas import tpu as pltpu, tpu_sc as plsc

N = 10000
D = 128
E = 320000
G = 128

NC = 2
NS = 16
NW = NC * NS
CH = 128
CPT = 80
E_PAD = NW * CPT * CH
NPAD = 10112
RPT = NPAD // NS

ECT = E_PAD // NS // CH
H = 5000
NACC = 5120
RPTA = NACC // NS
SEG = 20736
DUMMY = H


def _sc_mesh():
    return plsc.VectorSubcoreMesh(core_axis_name="c", subcore_axis_name="s",
                                  num_cores=NC, num_subcores=NS)


def _sc_deg_body(dst_hbm, ones_hbm, zeros_hbm, out_hbm, ones_v, idx_v, acc, sem):
    cid = lax.axis_index("c")
    sid = lax.axis_index("s")
    wid = sid * NC + cid
    pltpu.sync_copy(zeros_hbm, acc.at[pl.ds(sid * RPT, RPT)])
    pltpu.sync_copy(ones_hbm, ones_v)
    plsc.subcore_barrier()
    idx0, idx1 = idx_v
    sem0, sem1 = sem

    @pl.loop(0, CPT // 2)
    def _pair(t):
        base = wid * (CPT * CH) + t * (2 * CH)
        pltpu.sync_copy(dst_hbm.at[pl.ds(base, CH)], idx0)
        d0 = pltpu.async_copy(ones_v, acc.at[idx0], sem0, add=True)
        pltpu.sync_copy(dst_hbm.at[pl.ds(base + CH, CH)], idx1)
        d1 = pltpu.async_copy(ones_v, acc.at[idx1], sem1, add=True)
        d0.wait()
        d1.wait()

    plsc.subcore_barrier()
    pltpu.sync_copy(acc.at[pl.ds(sid * RPT, RPT)],
                    out_hbm.at[cid, pl.ds(sid * RPT, RPT)])


def _sc_deg(dst_pad, onesD, zerosD):
    return pl.kernel(
        _sc_deg_body,
        out_type=jax.ShapeDtypeStruct((NC, NPAD, D), jnp.float32),
        mesh=_sc_mesh(),
        scratch_types=[
            pltpu.VMEM((CH, D), jnp.float32),
            [pltpu.VMEM((CH,), jnp.int32)] * 2,
            pltpu.VMEM_SHARED((NPAD, D), jnp.float32),
            [pltpu.SemaphoreType.DMA] * 2,
        ],
    )(dst_pad, onesD, zerosD)


def _sc_part_body(src_hbm, dst_hbm, osrc_hbm, odst_hbm, ocnt_hbm,
                  ibs, ibd, sb0s, sb0d, sb1s, sb1d, cb):
    cid = lax.axis_index("c")
    sid = lax.axis_index("s")
    lo = cid * H

    @pl.loop(0, ECT, init_carry=(jnp.int32(0), jnp.int32(0)))
    def _scan(t, carry):
        cur0, cur1 = carry
        base = (sid * ECT + t) * CH
        pltpu.sync_copy(src_hbm.at[pl.ds(base, CH)], ibs)
        pltpu.sync_copy(dst_hbm.at[pl.ds(base, CH)], ibd)
        for j in range(8):
            s = ibs[pl.ds(j * 16, 16)]
            d = ibd[pl.ds(j * 16, 16)]
            mc = (d >= lo) & (d < lo + H)
            dl = d - lo
            m0 = mc & (s < H)
            m1 = mc & (s >= H)
            inc0 = plsc.cumsum(m0.astype(jnp.int32))
            inc1 = plsc.cumsum(m1.astype(jnp.int32))
            pos0 = cur0 + inc0 - 1
            pos1 = cur1 + inc1 - 1
            plsc.store_scatter(sb0s, [pos0], s, mask=m0)
            plsc.store_scatter(sb0d, [pos0], dl, mask=m0)
            plsc.store_scatter(sb1s, [pos1], s - H, mask=m1)
            plsc.store_scatter(sb1d, [pos1], dl, mask=m1)
            cur0 = cur0 + jnp.max(inc0)
            cur1 = cur1 + jnp.max(inc1)
        return cur0, cur1

    cur0, cur1 = _scan
    zero16 = jnp.zeros((16,), jnp.int32)
    dum16 = jnp.full((16,), DUMMY, jnp.int32)
    for j in range(16):
        sb0s[pl.ds(cur0 + j * 16, 16)] = zero16
        sb0d[pl.ds(cur0 + j * 16, 16)] = dum16
        sb1s[pl.ds(cur1 + j * 16, 16)] = zero16
        sb1d[pl.ds(cur1 + j * 16, 16)] = dum16
    pltpu.sync_copy(sb0s, osrc_hbm.at[cid, 0, sid])
    pltpu.sync_copy(sb0d, odst_hbm.at[cid, 0, sid])
    pltpu.sync_copy(sb1s, osrc_hbm.at[cid, 1, sid])
    pltpu.sync_copy(sb1d, odst_hbm.at[cid, 1, sid])
    cb[...] = jnp.full((16,), 0, jnp.int32) + cur0
    pltpu.sync_copy(cb, ocnt_hbm.at[cid, 0, sid])
    cb[...] = jnp.full((16,), 0, jnp.int32) + cur1
    pltpu.sync_copy(cb, ocnt_hbm.at[cid, 1, sid])


def _sc_part(src_pad, dst_pad):
    return pl.kernel(
        _sc_part_body,
        out_type=[jax.ShapeDtypeStruct((NC, 2, NS, SEG), jnp.int32),
                  jax.ShapeDtypeStruct((NC, 2, NS, SEG), jnp.int32),
                  jax.ShapeDtypeStruct((NC, 2, NS, 16), jnp.int32)],
        mesh=_sc_mesh(),
        scratch_types=[
            pltpu.VMEM((CH,), jnp.int32),
            pltpu.VMEM((CH,), jnp.int32),
            pltpu.VMEM((SEG,), jnp.int32),
            pltpu.VMEM((SEG,), jnp.int32),
            pltpu.VMEM((SEG,), jnp.int32),
            pltpu.VMEM((SEG,), jnp.int32),
            pltpu.VMEM((16,), jnp.int32),
        ],
        compiler_params=pltpu.CompilerParams(needs_layout_passes=False),
    )(src_pad, dst_pad)


def _sc_spmm_body(g_hbm, osrc_hbm, odst_hbm, ocnt_hbm, zeros_hbm, out_hbm,
                  idx_s, idx_d, rows, cb, gtab, acc, sems):
    cid = lax.axis_index("c")
    sid = lax.axis_index("s")
    pltpu.sync_copy(zeros_hbm, acc.at[pl.ds(sid * RPTA, RPTA)])

    for p in (0, 1):
        @pl.when(sid == 0)
        def _stage(p=p):
            pltpu.sync_copy(g_hbm.at[pl.ds(p * H, H)], gtab)

        plsc.subcore_barrier()
        pltpu.sync_copy(ocnt_hbm.at[cid, p, sid], cb)
        cnt = jnp.max(cb[...])
        npairs = (cnt + 255) // 256

        def start(b, c, p=p):
            pltpu.sync_copy(osrc_hbm.at[cid, p, sid, pl.ds(c * CH, CH)],
                            idx_s[b])
            pltpu.async_copy(gtab.at[idx_s[b]], rows[b], sems[b])

        def finish(b, c, p=p):
            pltpu.sync_copy(odst_hbm.at[cid, p, sid, pl.ds(c * CH, CH)],
                            idx_d[b])
            pltpu.make_async_copy(gtab.at[idx_s[b]], rows[b], sems[b]).wait()
            pltpu.sync_copy(rows[b], acc.at[idx_d[b]], add=True)

        @pl.when(npairs > 0)
        def _prime():
            start(0, 0)

        @pl.loop(0, npairs)
        def _pair(t):
            c = t * 2
            start(1, c + 1)
            finish(0, c)

            @pl.when(t < npairs - 1)
            def _pref():
                start(0, c + 2)

            finish(1, c + 1)

        plsc.subcore_barrier()

    pltpu.sync_copy(acc.at[pl.ds(sid * RPTA, RPTA)],
                    out_hbm.at[cid, pl.ds(sid * RPTA, RPTA)])


def _sc_spmm(g, osrc, odst, ocnt, zerosA):
    return pl.kernel(
        _sc_spmm_body,
        out_type=jax.ShapeDtypeStruct((NC, NACC, D), jnp.float32),
        mesh=_sc_mesh(),
        scratch_types=[
            [pltpu.VMEM((CH,), jnp.int32)] * 2,
            [pltpu.VMEM((CH,), jnp.int32)] * 2,
            [pltpu.VMEM((CH, D), jnp.float32)] * 2,
            pltpu.VMEM((16,), jnp.int32),
            pltpu.VMEM_SHARED((H, D), jnp.float32),
            pltpu.VMEM_SHARED((NACC, D), jnp.float32),
            [pltpu.SemaphoreType.DMA] * 2,
        ],
        compiler_params=pltpu.CompilerParams(needs_layout_passes=False),
    )(g, osrc, odst, ocnt, zerosA)


_BLK = 1000
_GRID = N // _BLK

_row_spec = pl.BlockSpec((_BLK, D), lambda i: (i, 0))
_col_spec = pl.BlockSpec((_BLK, 1), lambda i: (i, 0))
_w_spec = pl.BlockSpec((D, D), lambda i: (0, 0))
_b_spec = pl.BlockSpec((1, D), lambda i: (0, 0))
_s_spec = pl.BlockSpec((1, _BLK, D), lambda i: (i // 5, i % 5, 0))


def _tc1_body(x_ref, w_ref, dega_ref, degb_ref, g_ref, dinv_ref):
    deg = dega_ref[...] + degb_ref[...] + 1.0
    dinv = lax.rsqrt(deg)
    dinv_ref[...] = dinv
    h = jnp.dot(x_ref[...], w_ref[...], preferred_element_type=jnp.float32)
    g_ref[...] = dinv * h


def _tc1(x, W1, dega, degb):
    return pl.pallas_call(
        _tc1_body,
        grid=(_GRID,),
        in_specs=[_row_spec, _w_spec, _col_spec, _col_spec],
        out_specs=[_row_spec, _col_spec],
        out_shape=[jax.ShapeDtypeStruct((N, D), jnp.float32),
                   jax.ShapeDtypeStruct((N, 1), jnp.float32)],
    )(x, W1, dega, degb)


def _tc_mid_body(s_ref, g_ref, dinv_ref, b_ref, w_ref, gout_ref):
    dinv = dinv_ref[...]
    t = dinv * (s_ref[0] + g_ref[...]) + b_ref[...]
    t = jnp.maximum(t, 0.0)
    gout_ref[...] = dinv * jnp.dot(t, w_ref[...],
                                   preferred_element_type=jnp.float32)


def _tc_mid(s, g, dinv, b, Wn):
    return pl.pallas_call(
        _tc_mid_body,
        grid=(_GRID,),
        in_specs=[_s_spec, _row_spec, _col_spec, _b_spec, _w_spec],
        out_specs=_row_spec,
        out_shape=jax.ShapeDtypeStruct((N, D), jnp.float32),
    )(s, g, dinv, b, Wn)


def _tc_pool_body(s_ref, g_ref, dinv_ref, b_ref, batch_ref, out_ref,
                  accp, accc):
    i = pl.program_id(0)

    @pl.when(i == 0)
    def _init():
        accp[...] = jnp.zeros_like(accp)
        accc[...] = jnp.zeros_like(accc)

    h = dinv_ref[...] * (s_ref[0] + g_ref[...]) + b_ref[...]
    h = jnp.maximum(h, 0.0)
    gids = lax.broadcasted_iota(jnp.int32, (_BLK, G), 1)
    oh = (batch_ref[...] == gids).astype(jnp.float32)
    dims = (((0,), (0,)), ((), ()))
    accp[...] += lax.dot_general(oh, h, dims,
                                 preferred_element_type=jnp.float32)
    accc[...] += lax.dot_general(oh, jnp.ones_like(h), dims,
                                 preferred_element_type=jnp.float32)

    @pl.when(i == _GRID - 1)
    def _fin():
        out_ref[...] = accp[...] / jnp.maximum(accc[...], 1.0)


def _tc_pool(s, g, dinv, b, batch2):
    return pl.pallas_call(
        _tc_pool_body,
        grid=(_GRID,),
        in_specs=[_s_spec, _row_spec, _col_spec, _b_spec,
                  pl.BlockSpec((_BLK, 1), lambda i: (i, 0))],
        out_specs=pl.BlockSpec((G, D), lambda i: (0, 0)),
        out_shape=jax.ShapeDtypeStruct((G, D), jnp.float32),
        scratch_shapes=[pltpu.VMEM((G, D), jnp.float32),
                        pltpu.VMEM((G, D), jnp.float32)],
    )(s, g, dinv, b, batch2)


def kernel(x, edge_index, batch, W1, b1, W2, b2, W3, b3, Wl, bl):
    del Wl, bl
    src = edge_index[0].astype(jnp.int32)
    dst = edge_index[1].astype(jnp.int32)
    pad = E_PAD - E
    src_pad = jnp.concatenate([src, jnp.zeros((pad,), jnp.int32)])
    dst_pad = jnp.concatenate([dst, jnp.full((pad,), N, jnp.int32)])
    onesD = jnp.ones((CH, D), jnp.float32)
    zerosD = jnp.zeros((RPT, D), jnp.float32)
    zerosA = jnp.zeros((RPTA, D), jnp.float32)
    batch2 = batch.astype(jnp.int32)[:, None]

    deg = _sc_deg(dst_pad, onesD, zerosD)
    dega = deg[0, :N, 0:1]
    degb = deg[1, :N, 0:1]
    osrc, odst, ocnt = _sc_part(src_pad, dst_pad)

    g1, dinv = _tc1(x, W1, dega, degb)
    s1 = _sc_spmm(g1, osrc, odst, ocnt, zerosA)
    g2 = _tc_mid(s1, g1, dinv, b1[None, :], W2)
    s2 = _sc_spmm(g2, osrc, odst, ocnt, zerosA)
    g3 = _tc_mid(s2, g2, dinv, b2[None, :], W3)
    s3 = _sc_spmm(g3, osrc, odst, ocnt, zerosA)
    pooled = _tc_pool(s3, g3, dinv, b3[None, :], batch2)
    return pooled


# speedup vs baseline: 2.7581x; 1.1321x over previous
import jax
import jax.numpy as jnp
from jax import lax
from jax.experimental import pallas as pl
from jax.experimental.pallas import tpu as pltpu, tpu_sc as plsc

N = 10000
D = 128
E = 320000
G = 128

NC = 2
NS = 16
NW = NC * NS
CH = 128
CPT = 80
E_PAD = NW * CPT * CH
NPAD = 10112
RPT = NPAD // NS

ECT = E_PAD // NS // CH
H = 5000
NACC = 5120
RPTA = NACC // NS
SEG = 20736
DUMMY = H


def _sc_mesh():
    return plsc.VectorSubcoreMesh(core_axis_name="c", subcore_axis_name="s",
                                  num_cores=NC, num_subcores=NS)


def _sc_deg_body(dst_hbm, ones_hbm, zeros_hbm, out_hbm, ones_v, idx_v, acc, sem):
    cid = lax.axis_index("c")
    sid = lax.axis_index("s")
    wid = sid * NC + cid
    pltpu.sync_copy(zeros_hbm, acc.at[pl.ds(sid * RPT, RPT)])
    pltpu.sync_copy(ones_hbm, ones_v)
    plsc.subcore_barrier()
    idx0, idx1 = idx_v
    sem0, sem1 = sem

    @pl.loop(0, CPT // 2)
    def _pair(t):
        base = wid * (CPT * CH) + t * (2 * CH)
        pltpu.sync_copy(dst_hbm.at[pl.ds(base, CH)], idx0)
        d0 = pltpu.async_copy(ones_v, acc.at[idx0], sem0, add=True)
        pltpu.sync_copy(dst_hbm.at[pl.ds(base + CH, CH)], idx1)
        d1 = pltpu.async_copy(ones_v, acc.at[idx1], sem1, add=True)
        d0.wait()
        d1.wait()

    plsc.subcore_barrier()
    pltpu.sync_copy(acc.at[pl.ds(sid * RPT, RPT)],
                    out_hbm.at[cid, pl.ds(sid * RPT, RPT)])


def _sc_deg(dst_pad, onesD, zerosD):
    return pl.kernel(
        _sc_deg_body,
        out_type=jax.ShapeDtypeStruct((NC, NPAD, D), jnp.float32),
        mesh=_sc_mesh(),
        scratch_types=[
            pltpu.VMEM((CH, D), jnp.float32),
            [pltpu.VMEM((CH,), jnp.int32)] * 2,
            pltpu.VMEM_SHARED((NPAD, D), jnp.float32),
            [pltpu.SemaphoreType.DMA] * 2,
        ],
    )(dst_pad, onesD, zerosD)


_BCH = 8


def _sc_part_body(src_hbm, dst_hbm, osrc_hbm, odst_hbm, ocnt_hbm,
                  ibs, ibd, sb0s, sb0d, sb1s, sb1d, cb):
    cid = lax.axis_index("c")
    sid = lax.axis_index("s")
    lo = cid * H

    @pl.loop(0, ECT // _BCH, init_carry=(jnp.int32(0), jnp.int32(0)))
    def _scan(t, carry):
        cur0, cur1 = carry
        row = sid * ECT + t * _BCH
        pltpu.sync_copy(src_hbm.at[pl.ds(row, _BCH)], ibs)
        pltpu.sync_copy(dst_hbm.at[pl.ds(row, _BCH)], ibd)
        for k in range(_BCH):
            for j in range(8):
                s = ibs[k, pl.ds(j * 16, 16)]
                d = ibd[k, pl.ds(j * 16, 16)]
                mc = (d >= lo) & (d < lo + H)
                dl = d - lo
                m0 = mc & (s < H)
                m1 = mc & (s >= H)
                inc0 = plsc.cumsum(m0.astype(jnp.int32))
                inc1 = plsc.cumsum(m1.astype(jnp.int32))
                pos0 = cur0 + inc0 - 1
                pos1 = cur1 + inc1 - 1
                plsc.store_scatter(sb0s, [pos0], s, mask=m0)
                plsc.store_scatter(sb0d, [pos0], dl, mask=m0)
                plsc.store_scatter(sb1s, [pos1], s - H, mask=m1)
                plsc.store_scatter(sb1d, [pos1], dl, mask=m1)
                cur0 = cur0 + jnp.max(inc0)
                cur1 = cur1 + jnp.max(inc1)
        return cur0, cur1

    cur0, cur1 = _scan
    zero16 = jnp.zeros((16,), jnp.int32)
    dum16 = jnp.full((16,), DUMMY, jnp.int32)
    for j in range(16):
        sb0s[pl.ds(cur0 + j * 16, 16)] = zero16
        sb0d[pl.ds(cur0 + j * 16, 16)] = dum16
        sb1s[pl.ds(cur1 + j * 16, 16)] = zero16
        sb1d[pl.ds(cur1 + j * 16, 16)] = dum16
    pltpu.sync_copy(sb0s, osrc_hbm.at[cid, 0, sid])
    pltpu.sync_copy(sb0d, odst_hbm.at[cid, 0, sid])
    pltpu.sync_copy(sb1s, osrc_hbm.at[cid, 1, sid])
    pltpu.sync_copy(sb1d, odst_hbm.at[cid, 1, sid])
    cb[...] = jnp.full((16,), 0, jnp.int32) + cur0
    pltpu.sync_copy(cb, ocnt_hbm.at[cid, 0, sid])
    cb[...] = jnp.full((16,), 0, jnp.int32) + cur1
    pltpu.sync_copy(cb, ocnt_hbm.at[cid, 1, sid])


def _sc_part(src_pad, dst_pad):
    return pl.kernel(
        _sc_part_body,
        out_type=[jax.ShapeDtypeStruct((NC, 2, NS, SEG), jnp.int32),
                  jax.ShapeDtypeStruct((NC, 2, NS, SEG), jnp.int32),
                  jax.ShapeDtypeStruct((NC, 2, NS, 16), jnp.int32)],
        mesh=_sc_mesh(),
        scratch_types=[
            pltpu.VMEM((_BCH, CH), jnp.int32),
            pltpu.VMEM((_BCH, CH), jnp.int32),
            pltpu.VMEM((SEG,), jnp.int32),
            pltpu.VMEM((SEG,), jnp.int32),
            pltpu.VMEM((SEG,), jnp.int32),
            pltpu.VMEM((SEG,), jnp.int32),
            pltpu.VMEM((16,), jnp.int32),
        ],
        compiler_params=pltpu.CompilerParams(needs_layout_passes=False),
    )(src_pad.reshape(E_PAD // CH, CH), dst_pad.reshape(E_PAD // CH, CH))


def _sc_spmm_body(g_hbm, osrc_hbm, odst_hbm, ocnt_hbm, zeros_hbm, out_hbm,
                  idx_s, idx_d, rows, cb, gtab, acc, sems):
    cid = lax.axis_index("c")
    sid = lax.axis_index("s")
    pltpu.sync_copy(zeros_hbm, acc.at[pl.ds(sid * RPTA, RPTA)])

    for p in (0, 1):
        @pl.when(sid == 0)
        def _stage(p=p):
            pltpu.sync_copy(g_hbm.at[pl.ds(p * H, H)], gtab)

        plsc.subcore_barrier()
        pltpu.sync_copy(ocnt_hbm.at[cid, p, sid], cb)
        cnt = jnp.max(cb[...])
        npairs = (cnt + 255) // 256

        def start(b, c, p=p):
            pltpu.sync_copy(osrc_hbm.at[cid, p, sid, pl.ds(c * CH, CH)],
                            idx_s[b])
            pltpu.async_copy(gtab.at[idx_s[b]], rows[b], sems[b])

        def finish(b, c, p=p):
            pltpu.sync_copy(odst_hbm.at[cid, p, sid, pl.ds(c * CH, CH)],
                            idx_d[b])
            pltpu.make_async_copy(gtab.at[idx_s[b]], rows[b], sems[b]).wait()
            pltpu.sync_copy(rows[b], acc.at[idx_d[b]], add=True)

        @pl.when(npairs > 0)
        def _prime():
            start(0, 0)

        @pl.loop(0, npairs)
        def _pair(t):
            c = t * 2
            start(1, c + 1)
            finish(0, c)

            @pl.when(t < npairs - 1)
            def _pref():
                start(0, c + 2)

            finish(1, c + 1)

        plsc.subcore_barrier()

    pltpu.sync_copy(acc.at[pl.ds(sid * RPTA, RPTA)],
                    out_hbm.at[cid, pl.ds(sid * RPTA, RPTA)])


def _sc_spmm(g, osrc, odst, ocnt, zerosA):
    return pl.kernel(
        _sc_spmm_body,
        out_type=jax.ShapeDtypeStruct((NC, NACC, D), jnp.float32),
        mesh=_sc_mesh(),
        scratch_types=[
            [pltpu.VMEM((CH,), jnp.int32)] * 2,
            [pltpu.VMEM((CH,), jnp.int32)] * 2,
            [pltpu.VMEM((CH, D), jnp.float32)] * 2,
            pltpu.VMEM((16,), jnp.int32),
            pltpu.VMEM_SHARED((H, D), jnp.float32),
            pltpu.VMEM_SHARED((NACC, D), jnp.float32),
            [pltpu.SemaphoreType.DMA] * 2,
        ],
        compiler_params=pltpu.CompilerParams(needs_layout_passes=False),
    )(g, osrc, odst, ocnt, zerosA)


_BLK = 1000
_GRID = N // _BLK

_row_spec = pl.BlockSpec((_BLK, D), lambda i: (i, 0))
_col_spec = pl.BlockSpec((_BLK, 1), lambda i: (i, 0))
_w_spec = pl.BlockSpec((D, D), lambda i: (0, 0))
_b_spec = pl.BlockSpec((1, D), lambda i: (0, 0))
_s_spec = pl.BlockSpec((1, _BLK, D), lambda i: (i // 5, i % 5, 0))


def _tc1_body(x_ref, w_ref, dega_ref, degb_ref, g_ref, dinv_ref):
    deg = dega_ref[...] + degb_ref[...] + 1.0
    dinv = lax.rsqrt(deg)
    dinv_ref[...] = dinv
    h = jnp.dot(x_ref[...], w_ref[...], preferred_element_type=jnp.float32)
    g_ref[...] = dinv * h


def _tc1(x, W1, dega, degb):
    return pl.pallas_call(
        _tc1_body,
        grid=(_GRID,),
        in_specs=[_row_spec, _w_spec, _col_spec, _col_spec],
        out_specs=[_row_spec, _col_spec],
        out_shape=[jax.ShapeDtypeStruct((N, D), jnp.float32),
                   jax.ShapeDtypeStruct((N, 1), jnp.float32)],
    )(x, W1, dega, degb)


def _tc_mid_body(s_ref, g_ref, dinv_ref, b_ref, w_ref, gout_ref):
    dinv = dinv_ref[...]
    t = dinv * (s_ref[0] + g_ref[...]) + b_ref[...]
    t = jnp.maximum(t, 0.0)
    gout_ref[...] = dinv * jnp.dot(t, w_ref[...],
                                   preferred_element_type=jnp.float32)


def _tc_mid(s, g, dinv, b, Wn):
    return pl.pallas_call(
        _tc_mid_body,
        grid=(_GRID,),
        in_specs=[_s_spec, _row_spec, _col_spec, _b_spec, _w_spec],
        out_specs=_row_spec,
        out_shape=jax.ShapeDtypeStruct((N, D), jnp.float32),
    )(s, g, dinv, b, Wn)


def _tc_pool_body(s_ref, g_ref, dinv_ref, b_ref, batch_ref, out_ref,
                  accp, accc):
    i = pl.program_id(0)

    @pl.when(i == 0)
    def _init():
        accp[...] = jnp.zeros_like(accp)
        accc[...] = jnp.zeros_like(accc)

    h = dinv_ref[...] * (s_ref[0] + g_ref[...]) + b_ref[...]
    h = jnp.maximum(h, 0.0)
    gids = lax.broadcasted_iota(jnp.int32, (_BLK, G), 1)
    oh = (batch_ref[...] == gids).astype(jnp.float32)
    dims = (((0,), (0,)), ((), ()))
    accp[...] += lax.dot_general(oh, h, dims,
                                 preferred_element_type=jnp.float32)
    accc[...] += lax.dot_general(oh, jnp.ones_like(h), dims,
                                 preferred_element_type=jnp.float32)

    @pl.when(i == _GRID - 1)
    def _fin():
        out_ref[...] = accp[...] / jnp.maximum(accc[...], 1.0)


def _tc_pool(s, g, dinv, b, batch2):
    return pl.pallas_call(
        _tc_pool_body,
        grid=(_GRID,),
        in_specs=[_s_spec, _row_spec, _col_spec, _b_spec,
                  pl.BlockSpec((_BLK, 1), lambda i: (i, 0))],
        out_specs=pl.BlockSpec((G, D), lambda i: (0, 0)),
        out_shape=jax.ShapeDtypeStruct((G, D), jnp.float32),
        scratch_shapes=[pltpu.VMEM((G, D), jnp.float32),
                        pltpu.VMEM((G, D), jnp.float32)],
    )(s, g, dinv, b, batch2)


def kernel(x, edge_index, batch, W1, b1, W2, b2, W3, b3, Wl, bl):
    del Wl, bl
    src = edge_index[0].astype(jnp.int32)
    dst = edge_index[1].astype(jnp.int32)
    pad = E_PAD - E
    src_pad = jnp.concatenate([src, jnp.zeros((pad,), jnp.int32)])
    dst_pad = jnp.concatenate([dst, jnp.full((pad,), N, jnp.int32)])
    onesD = jnp.ones((CH, D), jnp.float32)
    zerosD = jnp.zeros((RPT, D), jnp.float32)
    zerosA = jnp.zeros((RPTA, D), jnp.float32)
    batch2 = batch.astype(jnp.int32)[:, None]

    deg = _sc_deg(dst_pad, onesD, zerosD)
    dega = deg[0, :N, 0:1]
    degb = deg[1, :N, 0:1]
    osrc, odst, ocnt = _sc_part(src_pad, dst_pad)

    g1, dinv = _tc1(x, W1, dega, degb)
    s1 = _sc_spmm(g1, osrc, odst, ocnt, zerosA)
    g2 = _tc_mid(s1, g1, dinv, b1[None, :], W2)
    s2 = _sc_spmm(g2, osrc, odst, ocnt, zerosA)
    g3 = _tc_mid(s2, g2, dinv, b2[None, :], W3)
    s3 = _sc_spmm(g3, osrc, odst, ocnt, zerosA)
    pooled = _tc_pool(s3, g3, dinv, b3[None, :], batch2)
    return pooled
